# Initial kernel scaffold; baseline (speedup 1.0000x reference)
import jax
import jax.numpy as jnp
from jax.experimental import pallas as pl


def kernel(x, segment_ids, params):
    raise NotImplementedError("write your pallas kernel here")


# speedup vs baseline: 3.6368x; 3.6368x over previous
import functools

import jax
import jax.numpy as jnp
from jax import lax
from jax.experimental import pallas as pl
from jax.experimental.pallas import tpu as pltpu

_NSEG = 512
_W = 64
_PAD = _NSEG + _W
_TILE = 2000
_EPS = 1e-5


def _win(ids_row_ref, starts_ref, nblk_ref, t, fn):
    a = starts_ref[t]
    nb = nblk_ref[t]
    idr = ids_row_ref[0]

    def body(k, carry):
        base = a + k * _W
        row = lax.broadcasted_iota(jnp.int32, (_W, _TILE), 0) + base
        oh_t = (row == idr).astype(jnp.float32)
        fn(base, oh_t)
        return carry

    lax.fori_loop(0, nb, body, 0)


def _segsum_body(ids_row_ref, x_ref, starts_ref, nblk_ref, s_ref):
    t = pl.program_id(0)

    @pl.when(t == 0)
    def _():
        s_ref[...] = jnp.zeros_like(s_ref)

    x = x_ref[...]

    def acc(base, oh_t):
        s_ref[pl.ds(base, _W), :] += jnp.dot(
            oh_t, x, preferred_element_type=jnp.float32)

    _win(ids_row_ref, starts_ref, nblk_ref, t, acc)


def _pass_a_body(ids_col_ref, h_ref, s_prev_ref,
                 pw1_ref, pb1_ref, pw2_ref, pb2_ref,
                 rw1_ref, rb1_ref, rw2_ref, rb2_ref,
                 starts_ref, nblk_ref,
                 y_ref, sum_ref, ssq_ref, rho_ref):
    t = pl.program_id(0)

    @pl.when(t == 0)
    def _():
        z = jnp.maximum(
            jnp.dot(s_prev_ref[...], rw1_ref[...],
                    preferred_element_type=jnp.float32) + rb1_ref[...], 0.0)
        rho_ref[...] = jnp.dot(
            z, rw2_ref[...], preferred_element_type=jnp.float32) + rb2_ref[...]
        sum_ref[...] = jnp.zeros_like(sum_ref)
        ssq_ref[...] = jnp.zeros_like(ssq_ref)

    h = h_ref[...]
    z1 = jnp.maximum(
        jnp.dot(h, pw1_ref[...], preferred_element_type=jnp.float32)
        + pb1_ref[...], 0.0)
    y_ref[...] = jnp.dot(
        z1, pw2_ref[...], preferred_element_type=jnp.float32) + pb2_ref[...]

    a = starts_ref[t]
    nb = nblk_ref[t]
    idc = ids_col_ref[0]

    def body(k, carry):
        base = a + k * _W
        col = lax.broadcasted_iota(jnp.int32, (_TILE, _W), 1) + base
        oh = (idc == col).astype(jnp.float32)
        y_ref[...] += jnp.dot(oh, rho_ref[pl.ds(base, _W), :],
                              preferred_element_type=jnp.float32)
        return carry

    lax.fori_loop(0, nb, body, 0)

    y = y_ref[...]
    sum_ref[...] += jnp.sum(y, axis=0, keepdims=True)
    ssq_ref[...] += jnp.sum(y * y, axis=0, keepdims=True)


def _bn_coeffs(sum_ref, ssq_ref, g_ref, b_ref, n_tokens):
    m = sum_ref[...] * (1.0 / n_tokens)
    v = ssq_ref[...] * (1.0 / n_tokens) - m * m
    scale = g_ref[...] * lax.rsqrt(v + _EPS)
    shift = b_ref[...] - m * scale
    return scale, shift


def _pass_b_body(n_tokens, ids_row_ref, h_ref, y_ref,
                 sum_ref, ssq_ref, g_ref, b_ref,
                 starts_ref, nblk_ref, ho_ref, s_ref):
    t = pl.program_id(0)

    @pl.when(t == 0)
    def _():
        s_ref[...] = jnp.zeros_like(s_ref)

    scale, shift = _bn_coeffs(sum_ref, ssq_ref, g_ref, b_ref, n_tokens)
    hn = h_ref[...] + y_ref[...] * scale + shift
    ho_ref[...] = hn

    def acc(base, oh_t):
        s_ref[pl.ds(base, _W), :] += jnp.dot(
            oh_t, hn, preferred_element_type=jnp.float32)

    _win(ids_row_ref, starts_ref, nblk_ref, t, acc)


def _pass_b_pool_body(n_tokens, ids_row_ref, h_ref, y_ref,
                      sum_ref, ssq_ref, g_ref, b_ref,
                      pw1_ref, pb1_ref, pw2_ref, pb2_ref,
                      starts_ref, nblk_ref, s_ref):
    t = pl.program_id(0)

    @pl.when(t == 0)
    def _():
        s_ref[...] = jnp.zeros_like(s_ref)

    scale, shift = _bn_coeffs(sum_ref, ssq_ref, g_ref, b_ref, n_tokens)
    hn = h_ref[...] + y_ref[...] * scale + shift
    z1 = jnp.maximum(
        jnp.dot(hn, pw1_ref[...], preferred_element_type=jnp.float32)
        + pb1_ref[...], 0.0)
    p = jnp.dot(z1, pw2_ref[...], preferred_element_type=jnp.float32) \
        + pb2_ref[...]

    def acc(base, oh_t):
        s_ref[pl.ds(base, _W), :] += jnp.dot(
            oh_t, p, preferred_element_type=jnp.float32)

    _win(ids_row_ref, starts_ref, nblk_ref, t, acc)


def _head_body(s_ref, w1_ref, b1_ref, w2_ref, b2_ref, o_ref):
    z = jnp.maximum(
        jnp.dot(s_ref[0:_NSEG, :], w1_ref[...],
                preferred_element_type=jnp.float32) + b1_ref[...], 0.0)
    o_ref[...] = jnp.dot(
        z, w2_ref[...], preferred_element_type=jnp.float32) + b2_ref[...]


def _full(shape):
    nd = len(shape)
    return pl.BlockSpec(shape, lambda t, _nd=nd: (0,) * _nd)


def _smem():
    return pl.BlockSpec(memory_space=pltpu.SMEM)


def kernel(x, segment_ids, params):
    n, d = x.shape
    n_tiles = n // _TILE
    ids = segment_ids.astype(jnp.int32)
    ids_row = ids.reshape(n_tiles, 1, _TILE)
    ids_col = ids.reshape(n_tiles, _TILE, 1)
    firsts = ids[::_TILE]
    lasts = ids[_TILE - 1::_TILE]
    starts = (firsts // 8) * 8
    nblk = (lasts - starts) // _W + 1

    grid = (n_tiles,)
    tok_spec = pl.BlockSpec((_TILE, d), lambda t: (t, 0))
    idr_spec = pl.BlockSpec((1, 1, _TILE), lambda t: (t, 0, 0))
    idc_spec = pl.BlockSpec((1, _TILE, 1), lambda t: (t, 0, 0))
    seg_shape = jax.ShapeDtypeStruct((_PAD, d), jnp.float32)
    vec_shape = jax.ShapeDtypeStruct((1, d), jnp.float32)

    def mats(mlp):
        w1 = mlp["l1"]["W"]
        b1 = mlp["l1"]["b"].reshape(1, -1)
        w2 = mlp["l2"]["W"]
        b2 = mlp["l2"]["b"].reshape(1, -1)
        return w1, b1, w2, b2

    def mat_specs(ms):
        return [_full(m.shape) for m in ms]

    s = pl.pallas_call(
        _segsum_body,
        grid=grid,
        in_specs=[idr_spec, tok_spec, _smem(), _smem()],
        out_specs=_full((_PAD, d)),
        out_shape=seg_shape,
    )(ids_row, x, starts, nblk)

    h = x
    n_layers = len(params["layers"])
    for li, lp in enumerate(params["layers"]):
        pm = mats(lp["phi"])
        rm = mats(lp["rho"])
        y, ysum, yssq = pl.pallas_call(
            _pass_a_body,
            grid=grid,
            in_specs=[idc_spec, tok_spec, _full((_PAD, d))]
            + mat_specs(pm) + mat_specs(rm) + [_smem(), _smem()],
            out_specs=[tok_spec, _full((1, d)), _full((1, d))],
            out_shape=[jax.ShapeDtypeStruct((n, d), jnp.float32),
                       vec_shape, vec_shape],
            scratch_shapes=[pltpu.VMEM((_PAD, d), jnp.float32)],
        )(ids_col, h, s, *pm, *rm, starts, nblk)

        g = lp["bn"]["gamma"].reshape(1, -1)
        b = lp["bn"]["beta"].reshape(1, -1)
        if li < n_layers - 1:
            h, s = pl.pallas_call(
                functools.partial(_pass_b_body, float(n)),
                grid=grid,
                in_specs=[idr_spec, tok_spec, tok_spec,
                          _full((1, d)), _full((1, d)),
                          _full((1, d)), _full((1, d)), _smem(), _smem()],
                out_specs=[tok_spec, _full((_PAD, d))],
                out_shape=[jax.ShapeDtypeStruct((n, d), jnp.float32),
                           seg_shape],
            )(ids_row, h, y, ysum, yssq, g, b, starts, nblk)
        else:
            qm = mats(params["pool"]["phi"])
            s = pl.pallas_call(
                functools.partial(_pass_b_pool_body, float(n)),
                grid=grid,
                in_specs=[idr_spec, tok_spec, tok_spec,
                          _full((1, d)), _full((1, d)),
                          _full((1, d)), _full((1, d))]
                + mat_specs(qm) + [_smem(), _smem()],
                out_specs=_full((_PAD, d)),
                out_shape=seg_shape,
            )(ids_row, h, y, ysum, yssq, g, b, *qm, starts, nblk)

    hm = mats(params["pool"]["rho"])
    dout = hm[2].shape[1]
    out = pl.pallas_call(
        _head_body,
        grid=(1,),
        in_specs=[_full((_PAD, d))] + mat_specs(hm),
        out_specs=_full((_NSEG, dout)),
        out_shape=jax.ShapeDtypeStruct((_NSEG, dout), jnp.float32),
    )(s, *hm)
    return out
